# baseline (speedup 1.0000x reference)
import functools
import jax
import jax.numpy as jnp
from jax import lax
from jax.experimental import pallas as pl
from jax.experimental.pallas import tpu as pltpu
from jax.experimental.pallas import tpu_sc as plsc

N = 10000
E = 320000
D_IN = 128
D_H = 64
D_OUT = 32

NC = 2
NS = 16
NW = NC * NS
EPW = E // NW
CHUNK = 80
NCHUNK = EPW // CHUNK
NPAD = 10240
RPT = NPAD // NS
RBLK = 128


NBUF = 5
NROUND = NCHUNK // NBUF


def _spmm_sc_body(x_hbm, row_hbm, col_hbm, w_hbm, out_hbm,
                  acc, col_all, row_all, w_all, rvs, obuf, gsems, ssems):
    c = lax.axis_index("c")
    s = lax.axis_index("s")
    wid = c * NS + s

    pltpu.sync_copy(col_hbm.at[wid], col_all)
    pltpu.sync_copy(row_hbm.at[wid], row_all)
    pltpu.sync_copy(w_hbm.at[wid], w_all)

    def _zrow(r, carry):
        for j in range(D_H // 16):
            obuf[r, pl.ds(j * 16, 16)] = jnp.zeros((16,), jnp.float32)
        return carry
    lax.fori_loop(0, RBLK, _zrow, 0)
    for k in range(RPT // RBLK):
        pltpu.sync_copy(obuf, acc.at[pl.ds(s * RPT + k * RBLK, RBLK), :])
    plsc.subcore_barrier()

    def _process(i, rv):
        def _grp(g, carry2):
            w16 = w_all[i, pl.ds(g * 16, 16)]
            for j in range(16):
                e = g * 16 + j
                wgt = w16[j]
                for t in range(D_H // 16):
                    seg = rv[e, pl.ds(t * 16, 16)]
                    rv[e, pl.ds(t * 16, 16)] = seg * wgt
            return carry2
        lax.fori_loop(0, CHUNK // 16, _grp, 0)

    for b in range(NBUF):
        pltpu.async_copy(x_hbm.at[col_all.at[b]], rvs[b], gsems[b])

    def _do_chunk(i, b):
        pltpu.make_async_copy(x_hbm.at[col_all.at[i]], rvs[b], gsems[b]).wait()
        _process(i, rvs[b])
        return pltpu.async_copy(rvs[b], acc.at[row_all.at[i]], ssems[b],
                                add=True)

    def _round(k, carry):
        base = k * NBUF
        descs = [_do_chunk(base + b, b) for b in range(NBUF)]
        for b in range(NBUF):
            descs[b].wait()
            pltpu.async_copy(x_hbm.at[col_all.at[base + NBUF + b]],
                             rvs[b], gsems[b])
        return carry
    lax.fori_loop(0, NROUND - 1, _round, 0)

    last = (NROUND - 1) * NBUF
    for b in range(NBUF):
        _do_chunk(last + b, b).wait()

    plsc.subcore_barrier()

    for k in range(RPT // RBLK):
        r0 = s * RPT + k * RBLK
        pltpu.sync_copy(acc.at[pl.ds(r0, RBLK), :], obuf)
        pltpu.sync_copy(obuf, out_hbm.at[c, pl.ds(r0, RBLK), :])


_spmm_sc = pl.kernel(
    _spmm_sc_body,
    out_type=jax.ShapeDtypeStruct((NC, NPAD, D_H), jnp.float32),
    mesh=plsc.VectorSubcoreMesh(core_axis_name="c", subcore_axis_name="s"),
    scratch_types=[
        pltpu.VMEM_SHARED((NPAD, D_H), jnp.float32),
        pltpu.VMEM((NCHUNK, CHUNK), jnp.int32),
        pltpu.VMEM((NCHUNK, CHUNK), jnp.int32),
        pltpu.VMEM((NCHUNK, CHUNK), jnp.float32),
        [pltpu.VMEM((CHUNK, D_H), jnp.float32) for _ in range(NBUF)],
        pltpu.VMEM((RBLK, D_H), jnp.float32),
        [pltpu.SemaphoreType.DMA for _ in range(NBUF)],
        [pltpu.SemaphoreType.DMA for _ in range(NBUF)],
    ],
    compiler_params=pltpu.CompilerParams(use_tc_tiling_on_sc=False),
)


def _mm_body(x_ref, w_ref, o_ref):
    o_ref[...] = jnp.dot(x_ref[...], w_ref[...],
                         preferred_element_type=jnp.float32)


def _mm(x, w):
    return pl.pallas_call(
        _mm_body,
        out_shape=jax.ShapeDtypeStruct((x.shape[0], w.shape[1]), jnp.float32),
    )(x, w)


def _relu_mm_body(a_ref, b_ref, w_ref, o_ref):
    h = jnp.maximum(a_ref[:N, :] + b_ref[:N, :], 0.0)
    o_ref[...] = jnp.dot(h, w_ref[...], preferred_element_type=jnp.float32)


def _relu_mm(a, b, w):
    return pl.pallas_call(
        _relu_mm_body,
        out_shape=jax.ShapeDtypeStruct((N, w.shape[1]), jnp.float32),
    )(a, b, w)


def _sample_body(a_ref, b_ref, eps_ref, zm_ref, zlv_ref, z_ref):
    s = a_ref[:N, :] + b_ref[:N, :]
    zm = s[:, :D_OUT]
    zlv = s[:, D_OUT:]
    zm_ref[...] = zm
    zlv_ref[...] = zlv
    z_ref[...] = zm + jnp.exp(0.5 * zlv) * eps_ref[...]


def _sample(a, b, eps):
    return pl.pallas_call(
        _sample_body,
        out_shape=(
            jax.ShapeDtypeStruct((N, D_OUT), jnp.float32),
            jax.ShapeDtypeStruct((N, D_OUT), jnp.float32),
            jax.ShapeDtypeStruct((N, D_OUT), jnp.float32),
        ),
    )(a, b, eps)


def kernel(x, edge_index, edge_weight, W0, Wm, Wv):
    row = edge_index[0].reshape(NW, NCHUNK, CHUNK)
    col = edge_index[1].reshape(NW, NCHUNK, CHUNK)
    ew = edge_weight.reshape(NW, NCHUNK, CHUNK)
    w_cat = jnp.concatenate([Wm, Wv], axis=1)
    eps = jax.random.normal(jax.random.key(42), (N, D_OUT), dtype=jnp.float32)

    p0 = _mm(x, W0)
    s1 = _spmm_sc(p0, row, col, ew)
    h1 = _relu_mm(s1[0], s1[1], w_cat)
    s2 = _spmm_sc(h1, row, col, ew)
    z_mean, z_log_var, z = _sample(s2[0], s2[1], eps)
    return (z_mean, z_log_var, z)


# speedup vs baseline: 1.8499x; 1.8499x over previous
import functools
import jax
import jax.numpy as jnp
from jax import lax
from jax.experimental import pallas as pl
from jax.experimental.pallas import tpu as pltpu
from jax.experimental.pallas import tpu_sc as plsc

N = 10000
E = 320000
D_IN = 128
D_H = 64
D_OUT = 32

NC = 2
NS = 16
NW = NC * NS
EPW = E // NW
CHUNK = 80
NCHUNK = EPW // CHUNK
NPAD = 10240
RPT = NPAD // NS
RBLK = 128


NBUF = 5
NROUND = NCHUNK // NBUF


def _spmm_sc_body(x_hbm, row_hbm, col_hbm, w_hbm, out_hbm,
                  acc, col_all, row_all, w_all, rvs, obuf, gsems, ssems):
    c = lax.axis_index("c")
    s = lax.axis_index("s")
    wid = c * NS + s

    pltpu.sync_copy(col_hbm.at[wid], col_all)
    pltpu.sync_copy(row_hbm.at[wid], row_all)
    pltpu.sync_copy(w_hbm.at[wid], w_all)

    def _zrow(r, carry):
        for j in range(D_H // 16):
            obuf[r, pl.ds(j * 16, 16)] = jnp.zeros((16,), jnp.float32)
        return carry
    lax.fori_loop(0, RBLK, _zrow, 0)
    for k in range(RPT // RBLK):
        pltpu.sync_copy(obuf, acc.at[pl.ds(s * RPT + k * RBLK, RBLK), :])
    plsc.subcore_barrier()

    def _process(i, rv):
        nseg = D_H // 16

        def _grp(g, carry2):
            w16 = w_all[i, pl.ds(g * 16, 16)]
            for j in range(0, 16, 2):
                e0 = g * 16 + j
                e1 = e0 + 1
                w0 = w16[j]
                w1 = w16[j + 1]
                segs = ([rv[e0, pl.ds(t * 16, 16)] for t in range(nseg)] +
                        [rv[e1, pl.ds(t * 16, 16)] for t in range(nseg)])
                outs = ([s * w0 for s in segs[:nseg]] +
                        [s * w1 for s in segs[nseg:]])
                for t in range(nseg):
                    rv[e0, pl.ds(t * 16, 16)] = outs[t]
                for t in range(nseg):
                    rv[e1, pl.ds(t * 16, 16)] = outs[nseg + t]
            return carry2
        lax.fori_loop(0, CHUNK // 16, _grp, 0)

    for b in range(NBUF):
        pltpu.async_copy(x_hbm.at[col_all.at[b]], rvs[b], gsems[b])

    def _do_chunk(i, b):
        pltpu.make_async_copy(x_hbm.at[col_all.at[i]], rvs[b], gsems[b]).wait()
        _process(i, rvs[b])
        return pltpu.async_copy(rvs[b], acc.at[row_all.at[i]], ssems[b],
                                add=True)

    def _round(k, carry):
        base = k * NBUF
        descs = [_do_chunk(base + b, b) for b in range(NBUF)]
        for b in range(NBUF):
            descs[b].wait()
            pltpu.async_copy(x_hbm.at[col_all.at[base + NBUF + b]],
                             rvs[b], gsems[b])
        return carry
    lax.fori_loop(0, NROUND - 1, _round, 0)

    last = (NROUND - 1) * NBUF
    for b in range(NBUF):
        _do_chunk(last + b, b).wait()

    plsc.subcore_barrier()

    for k in range(RPT // RBLK):
        r0 = s * RPT + k * RBLK
        pltpu.sync_copy(acc.at[pl.ds(r0, RBLK), :], obuf)
        pltpu.sync_copy(obuf, out_hbm.at[c, pl.ds(r0, RBLK), :])


_spmm_sc = pl.kernel(
    _spmm_sc_body,
    out_type=jax.ShapeDtypeStruct((NC, NPAD, D_H), jnp.float32),
    mesh=plsc.VectorSubcoreMesh(core_axis_name="c", subcore_axis_name="s"),
    scratch_types=[
        pltpu.VMEM_SHARED((NPAD, D_H), jnp.float32),
        pltpu.VMEM((NCHUNK, CHUNK), jnp.int32),
        pltpu.VMEM((NCHUNK, CHUNK), jnp.int32),
        pltpu.VMEM((NCHUNK, CHUNK), jnp.float32),
        [pltpu.VMEM((CHUNK, D_H), jnp.float32) for _ in range(NBUF)],
        pltpu.VMEM((RBLK, D_H), jnp.float32),
        [pltpu.SemaphoreType.DMA for _ in range(NBUF)],
        [pltpu.SemaphoreType.DMA for _ in range(NBUF)],
    ],
    compiler_params=pltpu.CompilerParams(use_tc_tiling_on_sc=False),
)


def _mm_body(x_ref, w_ref, o_ref):
    o_ref[...] = jnp.dot(x_ref[...], w_ref[...],
                         preferred_element_type=jnp.float32)


def _mm(x, w):
    return pl.pallas_call(
        _mm_body,
        out_shape=jax.ShapeDtypeStruct((x.shape[0], w.shape[1]), jnp.float32),
    )(x, w)


def _relu_mm_body(a_ref, b_ref, w_ref, o_ref):
    h = jnp.maximum(a_ref[:N, :] + b_ref[:N, :], 0.0)
    o_ref[...] = jnp.dot(h, w_ref[...], preferred_element_type=jnp.float32)


def _relu_mm(a, b, w):
    return pl.pallas_call(
        _relu_mm_body,
        out_shape=jax.ShapeDtypeStruct((N, w.shape[1]), jnp.float32),
    )(a, b, w)


def _sample_body(a_ref, b_ref, eps_ref, zm_ref, zlv_ref, z_ref):
    s = a_ref[:N, :] + b_ref[:N, :]
    zm = s[:, :D_OUT]
    zlv = s[:, D_OUT:]
    zm_ref[...] = zm
    zlv_ref[...] = zlv
    z_ref[...] = zm + jnp.exp(0.5 * zlv) * eps_ref[...]


def _sample(a, b, eps):
    return pl.pallas_call(
        _sample_body,
        out_shape=(
            jax.ShapeDtypeStruct((N, D_OUT), jnp.float32),
            jax.ShapeDtypeStruct((N, D_OUT), jnp.float32),
            jax.ShapeDtypeStruct((N, D_OUT), jnp.float32),
        ),
    )(a, b, eps)


def kernel(x, edge_index, edge_weight, W0, Wm, Wv):
    row = edge_index[0].reshape(NW, NCHUNK, CHUNK)
    col = edge_index[1].reshape(NW, NCHUNK, CHUNK)
    ew = edge_weight.reshape(NW, NCHUNK, CHUNK)
    w_cat = jnp.concatenate([Wm, Wv], axis=1)
    eps = jax.random.normal(jax.random.key(42), (N, D_OUT), dtype=jnp.float32)

    p0 = _mm(x, W0)
    s1 = _spmm_sc(p0, row, col, ew)
    h1 = _relu_mm(s1[0], s1[1], w_cat)
    s2 = _spmm_sc(h1, row, col, ew)
    z_mean, z_log_var, z = _sample(s2[0], s2[1], eps)
    return (z_mean, z_log_var, z)


# speedup vs baseline: 2.0372x; 1.1013x over previous
import functools
import jax
import jax.numpy as jnp
from jax import lax
from jax.experimental import pallas as pl
from jax.experimental.pallas import tpu as pltpu
from jax.experimental.pallas import tpu_sc as plsc

N = 10000
E = 320000
D_IN = 128
D_H = 64
D_OUT = 32

NC = 2
NS = 16
NW = NC * NS
EPW = E // NW
CHUNK = 80
NCHUNK = EPW // CHUNK
NPAD = 10240
RPT = NPAD // NS
RBLK = 128


NBUF = 5
NROUND = NCHUNK // NBUF


def _spmm_sc_body(x_hbm, row_hbm, col_hbm, w_hbm, out_hbm,
                  acc, col_all, row_all, w_all, rvs, obuf, gsems, ssems):
    c = lax.axis_index("c")
    s = lax.axis_index("s")
    wid = c * NS + s

    d_col = pltpu.async_copy(col_hbm.at[wid], col_all, gsems[0])
    d_row = pltpu.async_copy(row_hbm.at[wid], row_all, gsems[1])
    d_w = pltpu.async_copy(w_hbm.at[wid], w_all, gsems[2])

    def _zrow(r, carry):
        for j in range(D_H // 16):
            obuf[r, pl.ds(j * 16, 16)] = jnp.zeros((16,), jnp.float32)
        return carry
    lax.fori_loop(0, RBLK, _zrow, 0)
    for k in range(RPT // RBLK):
        pltpu.sync_copy(obuf, acc.at[pl.ds(s * RPT + k * RBLK, RBLK), :])
    d_col.wait()
    d_row.wait()
    d_w.wait()
    plsc.subcore_barrier()

    def _process(i, rv):
        nseg = D_H // 16

        def _grp(g, carry2):
            w16 = w_all[i, pl.ds(g * 16, 16)]
            for j in range(0, 16, 4):
                es = [g * 16 + j + u for u in range(4)]
                ws = [w16[j + u] for u in range(4)]
                segs = [rv[e, pl.ds(t * 16, 16)]
                        for e in es for t in range(nseg)]
                outs = [segs[u * nseg + t] * ws[u]
                        for u in range(4) for t in range(nseg)]
                for u in range(4):
                    for t in range(nseg):
                        rv[es[u], pl.ds(t * 16, 16)] = outs[u * nseg + t]
            return carry2
        lax.fori_loop(0, CHUNK // 16, _grp, 0)

    for b in range(NBUF):
        pltpu.async_copy(x_hbm.at[col_all.at[b]], rvs[b], gsems[b])

    def _do_chunk(i, b):
        pltpu.make_async_copy(x_hbm.at[col_all.at[i]], rvs[b], gsems[b]).wait()
        _process(i, rvs[b])
        return pltpu.async_copy(rvs[b], acc.at[row_all.at[i]], ssems[b],
                                add=True)

    def _round(k, carry):
        base = k * NBUF
        descs = [_do_chunk(base + b, b) for b in range(NBUF)]
        for b in range(NBUF):
            descs[b].wait()
            pltpu.async_copy(x_hbm.at[col_all.at[base + NBUF + b]],
                             rvs[b], gsems[b])
        return carry
    lax.fori_loop(0, NROUND - 1, _round, 0)

    last = (NROUND - 1) * NBUF
    for b in range(NBUF):
        _do_chunk(last + b, b).wait()

    plsc.subcore_barrier()

    r0 = s * RPT
    pltpu.sync_copy(acc.at[pl.ds(r0, RPT), :], out_hbm.at[c, pl.ds(r0, RPT), :])


_spmm_sc = pl.kernel(
    _spmm_sc_body,
    out_type=jax.ShapeDtypeStruct((NC, NPAD, D_H), jnp.float32),
    mesh=plsc.VectorSubcoreMesh(core_axis_name="c", subcore_axis_name="s"),
    scratch_types=[
        pltpu.VMEM_SHARED((NPAD, D_H), jnp.float32),
        pltpu.VMEM((NCHUNK, CHUNK), jnp.int32),
        pltpu.VMEM((NCHUNK, CHUNK), jnp.int32),
        pltpu.VMEM((NCHUNK, CHUNK), jnp.float32),
        [pltpu.VMEM((CHUNK, D_H), jnp.float32) for _ in range(NBUF)],
        pltpu.VMEM((RBLK, D_H), jnp.float32),
        [pltpu.SemaphoreType.DMA for _ in range(NBUF)],
        [pltpu.SemaphoreType.DMA for _ in range(NBUF)],
    ],
    compiler_params=pltpu.CompilerParams(use_tc_tiling_on_sc=False),
)


def _mm_body(x_ref, w_ref, o_ref):
    o_ref[...] = jnp.dot(x_ref[...], w_ref[...],
                         preferred_element_type=jnp.float32)


def _mm(x, w):
    return pl.pallas_call(
        _mm_body,
        out_shape=jax.ShapeDtypeStruct((x.shape[0], w.shape[1]), jnp.float32),
    )(x, w)


def _relu_mm_body(a_ref, b_ref, w_ref, o_ref):
    h = jnp.maximum(a_ref[:N, :] + b_ref[:N, :], 0.0)
    o_ref[...] = jnp.dot(h, w_ref[...], preferred_element_type=jnp.float32)


def _relu_mm(a, b, w):
    return pl.pallas_call(
        _relu_mm_body,
        out_shape=jax.ShapeDtypeStruct((N, w.shape[1]), jnp.float32),
    )(a, b, w)


def _sample_body(a_ref, b_ref, eps_ref, zm_ref, zlv_ref, z_ref):
    s = a_ref[:N, :] + b_ref[:N, :]
    zm = s[:, :D_OUT]
    zlv = s[:, D_OUT:]
    zm_ref[...] = zm
    zlv_ref[...] = zlv
    z_ref[...] = zm + jnp.exp(0.5 * zlv) * eps_ref[...]


def _sample(a, b, eps):
    return pl.pallas_call(
        _sample_body,
        out_shape=(
            jax.ShapeDtypeStruct((N, D_OUT), jnp.float32),
            jax.ShapeDtypeStruct((N, D_OUT), jnp.float32),
            jax.ShapeDtypeStruct((N, D_OUT), jnp.float32),
        ),
    )(a, b, eps)


def kernel(x, edge_index, edge_weight, W0, Wm, Wv):
    row = edge_index[0].reshape(NW, NCHUNK, CHUNK)
    col = edge_index[1].reshape(NW, NCHUNK, CHUNK)
    ew = edge_weight.reshape(NW, NCHUNK, CHUNK)
    w_cat = jnp.concatenate([Wm, Wv], axis=1)
    eps = jax.random.normal(jax.random.key(42), (N, D_OUT), dtype=jnp.float32)

    p0 = _mm(x, W0)
    s1 = _spmm_sc(p0, row, col, ew)
    h1 = _relu_mm(s1[0], s1[1], w_cat)
    s2 = _spmm_sc(h1, row, col, ew)
    z_mean, z_log_var, z = _sample(s2[0], s2[1], eps)
    return (z_mean, z_log_var, z)


# speedup vs baseline: 2.2434x; 1.1012x over previous
import functools
import jax
import jax.numpy as jnp
from jax import lax
from jax.experimental import pallas as pl
from jax.experimental.pallas import tpu as pltpu
from jax.experimental.pallas import tpu_sc as plsc

N = 10000
E = 320000
D_IN = 128
D_H = 64
D_OUT = 32

NC = 2
NS = 16
NW = NC * NS
EPW = E // NW
CHUNK = 80
NCHUNK = EPW // CHUNK
NPAD = 10240
RPT = NPAD // NS
RBLK = 128


NBUF = 5
NROUND = NCHUNK // NBUF


def _spmm_sc_body(x_hbm, row_hbm, col_hbm, w_hbm, out_hbm,
                  acc, col_all, row_all, w_all, rvs, obuf, gsems, ssems):
    c = lax.axis_index("c")
    s = lax.axis_index("s")
    wid = c * NS + s

    d_col = pltpu.async_copy(col_hbm.at[wid], col_all, gsems[0])
    d_row = pltpu.async_copy(row_hbm.at[wid], row_all, gsems[1])
    d_w = pltpu.async_copy(w_hbm.at[wid], w_all, gsems[2])

    def _zrow(r, carry):
        for j in range(D_H // 16):
            obuf[r, pl.ds(j * 16, 16)] = jnp.zeros((16,), jnp.float32)
        return carry
    lax.fori_loop(0, RBLK, _zrow, 0)
    for k in range(RPT // RBLK):
        pltpu.sync_copy(obuf, acc.at[pl.ds(s * RPT + k * RBLK, RBLK), :])
    d_col.wait()
    d_row.wait()
    d_w.wait()
    plsc.subcore_barrier()

    def _process(i, rv):
        nseg = D_H // 16

        def _grp(g, carry2):
            w16 = w_all[i, pl.ds(g * 16, 16)]
            for j in range(0, 16, 4):
                es = [g * 16 + j + u for u in range(4)]
                ws = [w16[j + u] for u in range(4)]
                segs = [rv[e, pl.ds(t * 16, 16)]
                        for e in es for t in range(nseg)]
                outs = [segs[u * nseg + t] * ws[u]
                        for u in range(4) for t in range(nseg)]
                for u in range(4):
                    for t in range(nseg):
                        rv[es[u], pl.ds(t * 16, 16)] = outs[u * nseg + t]
            return carry2
        lax.fori_loop(0, CHUNK // 16, _grp, 0)

    for b in range(NBUF):
        pltpu.async_copy(x_hbm.at[col_all.at[b]], rvs[b], gsems[b])

    def _do_chunk(i, b):
        pltpu.make_async_copy(x_hbm.at[col_all.at[i]], rvs[b], gsems[b]).wait()
        _process(i, rvs[b])
        return pltpu.async_copy(rvs[b], acc.at[row_all.at[i]], ssems[b],
                                add=True)

    LAG = 2

    def _round(k, carry):
        base = k * NBUF
        descs = [None] * NBUF
        for b in range(NBUF):
            descs[b] = _do_chunk(base + b, b)
            if b >= LAG:
                bb = b - LAG
                descs[bb].wait()
                pltpu.async_copy(x_hbm.at[col_all.at[base + NBUF + bb]],
                                 rvs[bb], gsems[bb])
        for bb in range(NBUF - LAG, NBUF):
            descs[bb].wait()
            pltpu.async_copy(x_hbm.at[col_all.at[base + NBUF + bb]],
                             rvs[bb], gsems[bb])
        return carry
    lax.fori_loop(0, NROUND - 1, _round, 0)

    last = (NROUND - 1) * NBUF
    for b in range(NBUF):
        _do_chunk(last + b, b).wait()

    plsc.subcore_barrier()

    r0 = s * RPT
    pltpu.sync_copy(acc.at[pl.ds(r0, RPT), :], out_hbm.at[c, pl.ds(r0, RPT), :])


_spmm_sc = pl.kernel(
    _spmm_sc_body,
    out_type=jax.ShapeDtypeStruct((NC, NPAD, D_H), jnp.float32),
    mesh=plsc.VectorSubcoreMesh(core_axis_name="c", subcore_axis_name="s"),
    scratch_types=[
        pltpu.VMEM_SHARED((NPAD, D_H), jnp.float32),
        pltpu.VMEM((NCHUNK, CHUNK), jnp.int32),
        pltpu.VMEM((NCHUNK, CHUNK), jnp.int32),
        pltpu.VMEM((NCHUNK, CHUNK), jnp.float32),
        [pltpu.VMEM((CHUNK, D_H), jnp.float32) for _ in range(NBUF)],
        pltpu.VMEM((RBLK, D_H), jnp.float32),
        [pltpu.SemaphoreType.DMA for _ in range(NBUF)],
        [pltpu.SemaphoreType.DMA for _ in range(NBUF)],
    ],
    compiler_params=pltpu.CompilerParams(use_tc_tiling_on_sc=False),
)


def _mm_body(x_ref, w_ref, o_ref):
    o_ref[...] = jnp.dot(x_ref[...], w_ref[...],
                         preferred_element_type=jnp.float32)


def _mm(x, w):
    return pl.pallas_call(
        _mm_body,
        out_shape=jax.ShapeDtypeStruct((x.shape[0], w.shape[1]), jnp.float32),
    )(x, w)


def _relu_mm_body(a_ref, b_ref, w_ref, o_ref):
    h = jnp.maximum(a_ref[:N, :] + b_ref[:N, :], 0.0)
    o_ref[...] = jnp.dot(h, w_ref[...], preferred_element_type=jnp.float32)


def _relu_mm(a, b, w):
    return pl.pallas_call(
        _relu_mm_body,
        out_shape=jax.ShapeDtypeStruct((N, w.shape[1]), jnp.float32),
    )(a, b, w)


def _sample_body(a_ref, b_ref, eps_ref, zm_ref, zlv_ref, z_ref):
    s = a_ref[:N, :] + b_ref[:N, :]
    zm = s[:, :D_OUT]
    zlv = s[:, D_OUT:]
    zm_ref[...] = zm
    zlv_ref[...] = zlv
    z_ref[...] = zm + jnp.exp(0.5 * zlv) * eps_ref[...]


def _sample(a, b, eps):
    return pl.pallas_call(
        _sample_body,
        out_shape=(
            jax.ShapeDtypeStruct((N, D_OUT), jnp.float32),
            jax.ShapeDtypeStruct((N, D_OUT), jnp.float32),
            jax.ShapeDtypeStruct((N, D_OUT), jnp.float32),
        ),
    )(a, b, eps)


def kernel(x, edge_index, edge_weight, W0, Wm, Wv):
    row = edge_index[0].reshape(NW, NCHUNK, CHUNK)
    col = edge_index[1].reshape(NW, NCHUNK, CHUNK)
    ew = edge_weight.reshape(NW, NCHUNK, CHUNK)
    w_cat = jnp.concatenate([Wm, Wv], axis=1)
    eps = jax.random.normal(jax.random.key(42), (N, D_OUT), dtype=jnp.float32)

    p0 = _mm(x, W0)
    s1 = _spmm_sc(p0, row, col, ew)
    h1 = _relu_mm(s1[0], s1[1], w_cat)
    s2 = _spmm_sc(h1, row, col, ew)
    z_mean, z_log_var, z = _sample(s2[0], s2[1], eps)
    return (z_mean, z_log_var, z)


# speedup vs baseline: 2.3763x; 1.0592x over previous
import functools
import jax
import jax.numpy as jnp
from jax import lax
from jax.experimental import pallas as pl
from jax.experimental.pallas import tpu as pltpu
from jax.experimental.pallas import tpu_sc as plsc

N = 10000
E = 320000
D_IN = 128
D_H = 64
D_OUT = 32

NC = 2
NS = 16
NW = NC * NS
EPW = E // NW
CHUNK = 80
NCHUNK = EPW // CHUNK
NPAD = 10240
RPT = NPAD // NS
RBLK = 128


NBUF = 5
NROUND = NCHUNK // NBUF


def _spmm_sc_body(x_hbm, row_hbm, col_hbm, w_hbm, out_hbm,
                  acc, col_all, row_all, w_all, rvps, rvs, obuf, gsems, ssems):
    c = lax.axis_index("c")
    s = lax.axis_index("s")
    wid = c * NS + s

    d_col = pltpu.async_copy(col_hbm.at[wid], col_all, gsems[0])
    d_row = pltpu.async_copy(row_hbm.at[wid], row_all, gsems[1])
    d_w = pltpu.async_copy(w_hbm.at[wid], w_all, gsems[2])

    def _zrow(r, carry):
        for j in range(D_H // 16):
            obuf[r, pl.ds(j * 16, 16)] = jnp.zeros((16,), jnp.float32)
        return carry
    lax.fori_loop(0, RBLK, _zrow, 0)
    for k in range(RPT // RBLK):
        pltpu.sync_copy(obuf, acc.at[pl.ds(s * RPT + k * RBLK, RBLK), :])
    d_col.wait()
    d_row.wait()
    d_w.wait()
    plsc.subcore_barrier()

    def _process(i, rvp, rv):
        def _grp(g, carry2):
            w16 = w_all[i, pl.ds(g * 16, 16)]
            for j in range(0, 16, 4):
                es = [g * 16 + j + u for u in range(4)]
                ws = [w16[j + u] for u in range(4)]
                ps = [(rvp[e, pl.ds(0, 16)], rvp[e, pl.ds(16, 16)])
                      for e in es]
                ups = [(plsc.unpack(plsc.bitcast(p0, jnp.bfloat16),
                                    format=plsc.PackFormat.INTERLEAVED),
                        plsc.unpack(plsc.bitcast(p1, jnp.bfloat16),
                                    format=plsc.PackFormat.INTERLEAVED))
                       for (p0, p1) in ps]
                for u in range(4):
                    (a0, b0), (a1, b1) = ups[u]
                    e = es[u]
                    wgt = ws[u]
                    rv[e, pl.ds(0, 16)] = a0 * wgt
                    rv[e, pl.ds(16, 16)] = a1 * wgt
                    rv[e, pl.ds(32, 16)] = b0 * wgt
                    rv[e, pl.ds(48, 16)] = b1 * wgt
            return carry2
        lax.fori_loop(0, CHUNK // 16, _grp, 0)

    for b in range(NBUF):
        pltpu.async_copy(x_hbm.at[col_all.at[b]], rvps[b], gsems[b])

    def _do_chunk(i, b):
        pltpu.make_async_copy(x_hbm.at[col_all.at[i]], rvps[b],
                              gsems[b]).wait()
        _process(i, rvps[b], rvs[b])
        return pltpu.async_copy(rvs[b], acc.at[row_all.at[i]], ssems[b],
                                add=True)

    LAG = 2

    def _round(k, carry):
        base = k * NBUF
        descs = [None] * NBUF
        for b in range(NBUF):
            descs[b] = _do_chunk(base + b, b)
            if b >= LAG:
                bb = b - LAG
                descs[bb].wait()
                pltpu.async_copy(x_hbm.at[col_all.at[base + NBUF + bb]],
                                 rvps[bb], gsems[bb])
        for bb in range(NBUF - LAG, NBUF):
            descs[bb].wait()
            pltpu.async_copy(x_hbm.at[col_all.at[base + NBUF + bb]],
                             rvps[bb], gsems[bb])
        return carry
    lax.fori_loop(0, NROUND - 1, _round, 0)

    last = (NROUND - 1) * NBUF
    for b in range(NBUF):
        _do_chunk(last + b, b).wait()

    plsc.subcore_barrier()

    r0 = s * RPT
    pltpu.sync_copy(acc.at[pl.ds(r0, RPT), :], out_hbm.at[c, pl.ds(r0, RPT), :])


_spmm_sc = pl.kernel(
    _spmm_sc_body,
    out_type=jax.ShapeDtypeStruct((NC, NPAD, D_H), jnp.float32),
    mesh=plsc.VectorSubcoreMesh(core_axis_name="c", subcore_axis_name="s"),
    scratch_types=[
        pltpu.VMEM_SHARED((NPAD, D_H), jnp.float32),
        pltpu.VMEM((NCHUNK, CHUNK), jnp.int32),
        pltpu.VMEM((NCHUNK, CHUNK), jnp.int32),
        pltpu.VMEM((NCHUNK, CHUNK), jnp.float32),
        [pltpu.VMEM((CHUNK, D_H // 2), jnp.int32) for _ in range(NBUF)],
        [pltpu.VMEM((CHUNK, D_H), jnp.float32) for _ in range(NBUF)],
        pltpu.VMEM((RBLK, D_H), jnp.float32),
        [pltpu.SemaphoreType.DMA for _ in range(NBUF)],
        [pltpu.SemaphoreType.DMA for _ in range(NBUF)],
    ],
    compiler_params=pltpu.CompilerParams(use_tc_tiling_on_sc=False,
                                         needs_layout_passes=False),
)


def _pack_pairs(p):
    lo = jax.lax.bitcast_convert_type(
        p[:, :D_H // 2].astype(jnp.bfloat16), jnp.uint16).astype(jnp.uint32)
    hi = jax.lax.bitcast_convert_type(
        p[:, D_H // 2:].astype(jnp.bfloat16), jnp.uint16).astype(jnp.uint32)
    return jax.lax.bitcast_convert_type(lo | (hi << 16), jnp.int32)


def _mm_body(x_ref, w_ref, o_ref):
    p = jnp.dot(x_ref[...], w_ref[...], preferred_element_type=jnp.float32)
    o_ref[...] = _pack_pairs(p)


def _mm(x, w):
    return pl.pallas_call(
        _mm_body,
        out_shape=jax.ShapeDtypeStruct((x.shape[0], w.shape[1] // 2),
                                       jnp.int32),
    )(x, w)


def _relu_mm_body(a_ref, b_ref, w_ref, o_ref):
    h = jnp.maximum(a_ref[:N, :] + b_ref[:N, :], 0.0)
    p = jnp.dot(h, w_ref[...], preferred_element_type=jnp.float32)
    o_ref[...] = _pack_pairs(p)


def _relu_mm(a, b, w):
    return pl.pallas_call(
        _relu_mm_body,
        out_shape=jax.ShapeDtypeStruct((N, w.shape[1] // 2), jnp.int32),
    )(a, b, w)


def _sample_body(a_ref, b_ref, eps_ref, zm_ref, zlv_ref, z_ref):
    s = a_ref[:N, :] + b_ref[:N, :]
    zm = s[:, :D_OUT]
    zlv = s[:, D_OUT:]
    zm_ref[...] = zm
    zlv_ref[...] = zlv
    z_ref[...] = zm + jnp.exp(0.5 * zlv) * eps_ref[...]


def _sample(a, b, eps):
    return pl.pallas_call(
        _sample_body,
        out_shape=(
            jax.ShapeDtypeStruct((N, D_OUT), jnp.float32),
            jax.ShapeDtypeStruct((N, D_OUT), jnp.float32),
            jax.ShapeDtypeStruct((N, D_OUT), jnp.float32),
        ),
    )(a, b, eps)


def kernel(x, edge_index, edge_weight, W0, Wm, Wv):
    row = edge_index[0].reshape(NW, NCHUNK, CHUNK)
    col = edge_index[1].reshape(NW, NCHUNK, CHUNK)
    ew = edge_weight.reshape(NW, NCHUNK, CHUNK)
    w_cat = jnp.concatenate([Wm, Wv], axis=1)
    eps = jax.random.normal(jax.random.key(42), (N, D_OUT), dtype=jnp.float32)

    p0 = _mm(x, W0)
    s1 = _spmm_sc(p0, row, col, ew)
    h1 = _relu_mm(s1[0], s1[1], w_cat)
    s2 = _spmm_sc(h1, row, col, ew)
    z_mean, z_log_var, z = _sample(s2[0], s2[1], eps)
    return (z_mean, z_log_var, z)
